# baseline (speedup 1.0000x reference)
import functools

import jax
import jax.numpy as jnp
from jax.experimental import pallas as pl
from jax.experimental.pallas import tpu as pltpu

K1 = (1832780943, 270669613)
K2 = (64467757, 2916123636)
K3 = (2465931498, 255383827)
K5 = (1914800406, 1741898942)

_ROT_A = (13, 15, 26, 6)
_ROT_B = (17, 29, 16, 24)

H = 480
W = 640
P_REPLACE = 0.1

ROWS, COLS = 32, 32768
BLOCK_ROWS = 8


def _rotl(v, d):
    return (v << jnp.uint32(d)) | (v >> jnp.uint32(32 - d))


def _threefry_fold(key, cnt):
    ks0 = jnp.uint32(key[0])
    ks1 = jnp.uint32(key[1])
    ks2 = jnp.uint32(0x1BD11BDA ^ key[0] ^ key[1])
    x0 = jnp.full_like(cnt, ks0)
    x1 = cnt + ks1

    def rounds(x0, x1, rots):
        for r in rots:
            x0 = x0 + x1
            x1 = _rotl(x1, r)
            x1 = x0 ^ x1
        return x0, x1

    x0, x1 = rounds(x0, x1, _ROT_A)
    x0 = x0 + ks1
    x1 = x1 + (ks2 + jnp.uint32(1))
    x0, x1 = rounds(x0, x1, _ROT_B)
    x0 = x0 + ks2
    x1 = x1 + (ks0 + jnp.uint32(2))
    x0, x1 = rounds(x0, x1, _ROT_A)
    x0 = x0 + ks0
    x1 = x1 + (ks1 + jnp.uint32(3))
    x0, x1 = rounds(x0, x1, _ROT_B)
    x0 = x0 + ks1
    x1 = x1 + (ks2 + jnp.uint32(4))
    x0, x1 = rounds(x0, x1, _ROT_A)
    x0 = x0 + ks2
    x1 = x1 + (ks0 + jnp.uint32(5))
    return x0 ^ x1


def _bits_to_unit_float(bits):
    f = jax.lax.bitcast_convert_type(
        (bits >> jnp.uint32(9)) | jnp.uint32(0x3F800000), jnp.float32)
    return f - jnp.float32(1.0)


def _body(p_ref, y_ref, x_ref, v_ref, po_ref, yo_ref, xo_ref):
    shape = p_ref.shape
    base = jnp.uint32(pl.program_id(0) * (BLOCK_ROWS * COLS)
                      + pl.program_id(1) * (COLS // 8))
    row = jax.lax.broadcasted_iota(jnp.uint32, shape, 0)
    col = jax.lax.broadcasted_iota(jnp.uint32, shape, 1)
    cnt = (base + (row << jnp.uint32(15))) + col

    b1 = _threefry_fold(K1, cnt)
    mask = ((b1 >> jnp.uint32(9)).astype(jnp.int32) < jnp.int32(838861)) & v_ref[...]

    nx = _bits_to_unit_float(_threefry_fold(K2, cnt)) * jnp.float32(W - 1)
    xo_ref[...] = jnp.where(mask, nx, x_ref[...])

    ny = _bits_to_unit_float(_threefry_fold(K3, cnt)) * jnp.float32(H - 1)
    yo_ref[...] = jnp.where(mask, ny, y_ref[...])

    np_bit = (_threefry_fold(K5, cnt) & jnp.uint32(1)).astype(jnp.float32)
    po_ref[...] = jnp.where(mask, np_bit, p_ref[...])


@functools.partial(jax.jit, donate_argnums=())
def kernel(p, y, x, t, valid_mask):
    grid = (ROWS // BLOCK_ROWS, 8)
    spec = pl.BlockSpec((BLOCK_ROWS, COLS // 8), lambda i, j: (i, j))
    p_out, y_out, x_out = pl.pallas_call(
        _body,
        grid=grid,
        in_specs=[spec, spec, spec, spec],
        out_specs=[spec, spec, spec],
        out_shape=[
            jax.ShapeDtypeStruct((ROWS, COLS), jnp.float32),
            jax.ShapeDtypeStruct((ROWS, COLS), jnp.float32),
            jax.ShapeDtypeStruct((ROWS, COLS), jnp.float32),
        ],
        compiler_params=pltpu.CompilerParams(
            dimension_semantics=("parallel", "parallel"),
        ),
    )(p, y, x, valid_mask)
    return (p_out, y_out, x_out, t, valid_mask)


# speedup vs baseline: 1.0112x; 1.0112x over previous
import functools

import jax
import jax.numpy as jnp
from jax.experimental import pallas as pl
from jax.experimental.pallas import tpu as pltpu

K1 = (1832780943, 270669613)
K2 = (64467757, 2916123636)
K3 = (2465931498, 255383827)
K5 = (1914800406, 1741898942)

_ROT_A = (13, 15, 26, 6)
_ROT_B = (17, 29, 16, 24)

H = 480
W = 640
P_REPLACE = 0.1

ROWS, COLS = 32, 32768
BLOCK_ROWS = 8


def _rotl(v, d):
    return (v << jnp.uint32(d)) | (v >> jnp.uint32(32 - d))


def _threefry_fold(key, cnt):
    ks0 = jnp.uint32(key[0])
    ks1 = jnp.uint32(key[1])
    ks2 = jnp.uint32(0x1BD11BDA ^ key[0] ^ key[1])
    x0 = jnp.full_like(cnt, ks0)
    x1 = cnt + ks1

    def rounds(x0, x1, rots):
        for r in rots:
            x0 = x0 + x1
            x1 = _rotl(x1, r)
            x1 = x0 ^ x1
        return x0, x1

    x0, x1 = rounds(x0, x1, _ROT_A)
    x0 = x0 + ks1
    x1 = x1 + (ks2 + jnp.uint32(1))
    x0, x1 = rounds(x0, x1, _ROT_B)
    x0 = x0 + ks2
    x1 = x1 + (ks0 + jnp.uint32(2))
    x0, x1 = rounds(x0, x1, _ROT_A)
    x0 = x0 + ks0
    x1 = x1 + (ks1 + jnp.uint32(3))
    x0, x1 = rounds(x0, x1, _ROT_B)
    x0 = x0 + ks1
    x1 = x1 + (ks2 + jnp.uint32(4))
    x0, x1 = rounds(x0, x1, _ROT_A)
    x0 = x0 + ks2
    x1 = x1 + (ks0 + jnp.uint32(5))
    return x0 ^ x1


def _bits_to_unit_float(bits):
    f = jax.lax.bitcast_convert_type(
        (bits >> jnp.uint32(9)) | jnp.uint32(0x3F800000), jnp.float32)
    return f - jnp.float32(1.0)


def _body(p_ref, y_ref, x_ref, v_ref, po_ref, yo_ref, xo_ref):
    shape = p_ref.shape
    base = jnp.uint32(pl.program_id(0) * (BLOCK_ROWS * COLS)
                      + pl.program_id(1) * (COLS // 4))
    row = jax.lax.broadcasted_iota(jnp.uint32, shape, 0)
    col = jax.lax.broadcasted_iota(jnp.uint32, shape, 1)
    cnt = (base + (row << jnp.uint32(15))) + col

    b1 = _threefry_fold(K1, cnt)
    mask = ((b1 >> jnp.uint32(9)).astype(jnp.int32) < jnp.int32(838861)) & v_ref[...]

    nx = _bits_to_unit_float(_threefry_fold(K2, cnt)) * jnp.float32(W - 1)
    xo_ref[...] = jnp.where(mask, nx, x_ref[...])

    ny = _bits_to_unit_float(_threefry_fold(K3, cnt)) * jnp.float32(H - 1)
    yo_ref[...] = jnp.where(mask, ny, y_ref[...])

    np_bit = (_threefry_fold(K5, cnt) & jnp.uint32(1)).astype(jnp.float32)
    po_ref[...] = jnp.where(mask, np_bit, p_ref[...])


@functools.partial(jax.jit, donate_argnums=())
def kernel(p, y, x, t, valid_mask):
    grid = (ROWS // BLOCK_ROWS, 4)
    spec = pl.BlockSpec((BLOCK_ROWS, COLS // 4), lambda i, j: (i, j))
    p_out, y_out, x_out = pl.pallas_call(
        _body,
        grid=grid,
        in_specs=[spec, spec, spec, spec],
        out_specs=[spec, spec, spec],
        out_shape=[
            jax.ShapeDtypeStruct((ROWS, COLS), jnp.float32),
            jax.ShapeDtypeStruct((ROWS, COLS), jnp.float32),
            jax.ShapeDtypeStruct((ROWS, COLS), jnp.float32),
        ],
        compiler_params=pltpu.CompilerParams(
            dimension_semantics=("parallel", "parallel"),
        ),
    )(p, y, x, valid_mask)
    return (p_out, y_out, x_out, t, valid_mask)
